# baseline (speedup 1.0000x reference)
import jax
import jax.numpy as jnp
from jax import lax
from jax.experimental import pallas as pl
from jax.experimental.pallas import tpu as pltpu
from jax.experimental.pallas import tpu_sc as plsc

VOCAB = 100000
EMB = 128
MAX_LEN = 512
B = 4096
L = 200

NC = 2
NS = 16
NW = NC * NS
N = B * L
TOK_PER_W = N // NW
CHUNK = 128
NCHUNK = TOK_PER_W // CHUNK
NF = EMB // 16
NGRP = CHUNK // 16


def _allsum16(v):
    lanes = lax.iota(jnp.int32, 16)
    for k in (1, 2, 4, 8):
        v = v + v.at[lanes ^ k].get(mode="promise_in_bounds")
    return v


def _rsqrt16(v):
    bits = lax.bitcast_convert_type(v, jnp.int32)
    seed = lax.bitcast_convert_type(jnp.int32(0x5F3759DF) - (bits >> 1),
                                    jnp.float32)
    half = v * 0.5
    y = seed
    for _ in range(2):
        y = y * (1.5 - half * y * y)
    return y


def _body(src_hbm, seg_hbm, word_hbm, segtab_hbm, postab_hbm, gamma_hbm,
          beta_hbm, out_hbm, idx_v, segc_v, rows_v, combo_v, gb_v,
          isem0, isem1, isem2, isem3, gsem0, gsem1, osem0, osem1):
    wid = lax.axis_index("s") * NC + lax.axis_index("c")
    base = wid * TOK_PER_W
    isem = (isem0, isem1, isem2, isem3)
    gsem = (gsem0, gsem1)
    osem = (osem0, osem1)

    pltpu.sync_copy(postab_hbm.at[pl.ds(0, L)], combo_v.at[pl.ds(0, L)])
    pltpu.sync_copy(segtab_hbm, combo_v.at[pl.ds(3 * L, 3)])
    pltpu.sync_copy(gamma_hbm, gb_v.at[0])
    pltpu.sync_copy(beta_hbm, gb_v.at[1])

    def build_p(p, _):
        for s in (2, 1, 0):
            for f in range(NF):
                d = pl.ds(f * 16, 16)
                combo_v[p * 3 + s, d] = combo_v[p, d] + combo_v[3 * L + s, d]
        return 0
    lax.fori_loop(0, L, lambda i, c: build_p(L - 1 - i, c), 0)

    gamma = [gb_v[0, pl.ds(f * 16, 16)] for f in range(NF)]
    beta = [gb_v[1, pl.ds(f * 16, 16)] for f in range(NF)]

    def issue_i(c, s4):
        cb = base + c * CHUNK
        pltpu.async_copy(src_hbm.at[pl.ds(cb, CHUNK)], idx_v.at[s4], isem[s4])
        pltpu.async_copy(seg_hbm.at[pl.ds(cb, CHUNK)], segc_v.at[s4],
                         isem[s4])

    def wait_i(s4):
        pltpu.make_async_copy(src_hbm.at[pl.ds(0, CHUNK)], idx_v.at[s4],
                              isem[s4]).wait()
        pltpu.make_async_copy(seg_hbm.at[pl.ds(0, CHUNK)], segc_v.at[s4],
                              isem[s4]).wait()

    def issue_g(s4, b):
        pltpu.async_copy(word_hbm.at[idx_v.at[s4]], rows_v.at[b], gsem[b])

    def wait_g(s4, b):
        pltpu.make_async_copy(word_hbm.at[idx_v.at[s4]], rows_v.at[b],
                              gsem[b]).wait()

    def issue_o(c, b):
        cb = base + c * CHUNK
        pltpu.async_copy(rows_v.at[b], out_hbm.at[pl.ds(cb, CHUNK)], osem[b])

    def wait_o(b):
        pltpu.make_async_copy(rows_v.at[b], out_hbm.at[pl.ds(0, CHUNK)],
                              osem[b]).wait()

    def compute(c, b, s4):
        p0 = lax.rem(c * CHUNK, L)

        def grp(g, _):
            sv = segc_v[s4, pl.ds(g * 16, 16)]
            sums = []
            sqs = []
            for k in range(16):
                t = g * 16 + k
                s_t = sv[k]
                p_t = lax.rem(p0 + t, L)
                ci = p_t * 3 + s_t
                sum_v = None
                sq_v = None
                for f in range(NF):
                    d = pl.ds(f * 16, 16)
                    xf = rows_v[b, t, d] + combo_v[ci, d]
                    rows_v[b, t, d] = xf
                    sum_v = xf if sum_v is None else sum_v + xf
                    sq_v = xf * xf if sq_v is None else sq_v + xf * xf
                sums.append(sum_v)
                sqs.append(sq_v)
            means = []
            rstds = []
            for k in range(16):
                mean = _allsum16(sums[k]) * (1.0 / EMB)
                ex2 = _allsum16(sqs[k]) * (1.0 / EMB)
                var = ex2 - mean * mean
                means.append(mean)
                rstds.append(_rsqrt16(var + 1e-6))
            for k in range(16):
                t = g * 16 + k
                for f in range(NF):
                    d = pl.ds(f * 16, 16)
                    rg = rstds[k] * gamma[f]
                    off = beta[f] - means[k] * rg
                    rows_v[b, t, d] = rows_v[b, t, d] * rg + off
            return 0

        lax.fori_loop(0, NGRP, grp, 0)

    issue_i(0, 0)
    issue_i(1, 1)
    wait_i(0)
    issue_g(0, 0)

    def iter4(c4, _):
        for u in range(4):
            b = u % 2
            nb = 1 - b
            c = c4 * 4 + u
            wait_g(u, b)

            @pl.when(c + 2 < NCHUNK)
            def _():
                issue_i(c + 2, (u + 2) % 4)

            @pl.when(c + 1 < NCHUNK)
            def _():
                @pl.when(c >= 1)
                def _():
                    wait_o(nb)
                wait_i((u + 1) % 4)
                issue_g((u + 1) % 4, nb)

            compute(c, b, u)
            issue_o(c, b)
        return 0

    lax.fori_loop(0, NCHUNK // 4, iter4, 0)
    wait_o(0)
    wait_o(1)


@jax.jit
def _tab_embedding(src, seg, word_table, seg_table, pos_table, gamma, beta):
    mesh = plsc.VectorSubcoreMesh(core_axis_name="c", subcore_axis_name="s")
    kern = pl.kernel(
        _body,
        out_type=jax.ShapeDtypeStruct((N, EMB), jnp.float32),
        mesh=mesh,
        scratch_types=[
            pltpu.VMEM((4, CHUNK), jnp.int32),
            pltpu.VMEM((4, CHUNK), jnp.int32),
            pltpu.VMEM((2, CHUNK, EMB), jnp.float32),
            pltpu.VMEM((3 * L + 3, EMB), jnp.float32),
            pltpu.VMEM((2, EMB), jnp.float32),
            pltpu.SemaphoreType.DMA,
            pltpu.SemaphoreType.DMA,
            pltpu.SemaphoreType.DMA,
            pltpu.SemaphoreType.DMA,
            pltpu.SemaphoreType.DMA,
            pltpu.SemaphoreType.DMA,
            pltpu.SemaphoreType.DMA,
            pltpu.SemaphoreType.DMA,
        ],
    )
    out = kern(src.reshape(N), seg.reshape(N), word_table, seg_table,
               pos_table, gamma, beta)
    return out.reshape(B, L, EMB)


def kernel(src, seg, word_table, seg_table, pos_table, gamma, beta):
    return _tab_embedding(src, seg, word_table, seg_table, pos_table,
                          gamma, beta)


# speedup vs baseline: 1.0889x; 1.0889x over previous
import functools

import jax
import jax.numpy as jnp
from jax import lax
from jax.experimental import pallas as pl
from jax.experimental.pallas import tpu as pltpu
from jax.experimental.pallas import tpu_sc as plsc

VOCAB = 100000
EMB = 128
MAX_LEN = 512
B = 4096
L = 200

NC = 2
NS = 16
NW = NC * NS
N = B * L
TOK_PER_W = N // NW
CHUNK = 128
NCHUNK = TOK_PER_W // CHUNK
NF = EMB // 16
NGRP = CHUNK // 16


def _allsum16(v):
    lanes = lax.iota(jnp.int32, 16)
    for k in (1, 2, 4, 8):
        v = v + v.at[lanes ^ k].get(mode="promise_in_bounds")
    return v


def _rsqrt16(v):
    bits = lax.bitcast_convert_type(v, jnp.int32)
    seed = lax.bitcast_convert_type(jnp.int32(0x5F3759DF) - (bits >> 1),
                                    jnp.float32)
    half = v * 0.5
    y = seed
    for _ in range(2):
        y = y * (1.5 - half * y * y)
    return y


def _body(src_hbm, seg_hbm, word_hbm, segtab_hbm, postab_hbm, gamma_hbm,
          beta_hbm, out_hbm, idx_v, segc_v, rows_v, combo_v, gb_v,
          isem0, isem1, isem2, isem3, gsem0, gsem1, osem0, osem1):
    wid = lax.axis_index("s") * NC + lax.axis_index("c")
    base = wid * TOK_PER_W
    isem = (isem0, isem1, isem2, isem3)
    gsem = (gsem0, gsem1)
    osem = (osem0, osem1)

    pltpu.sync_copy(postab_hbm.at[pl.ds(0, L)], combo_v.at[pl.ds(0, L)])
    pltpu.sync_copy(segtab_hbm, combo_v.at[pl.ds(3 * L, 3)])
    pltpu.sync_copy(gamma_hbm, gb_v.at[0])
    pltpu.sync_copy(beta_hbm, gb_v.at[1])

    def build_p(p, _):
        for s in (2, 1, 0):
            for f in range(NF):
                d = pl.ds(f * 16, 16)
                combo_v[p * 3 + s, d] = combo_v[p, d] + combo_v[3 * L + s, d]
        return 0
    lax.fori_loop(0, L, lambda i, c: build_p(L - 1 - i, c), 0)

    gamma = [gb_v[0, pl.ds(f * 16, 16)] for f in range(NF)]
    beta = [gb_v[1, pl.ds(f * 16, 16)] for f in range(NF)]

    def issue_i(c, s4):
        cb = base + c * CHUNK
        pltpu.async_copy(src_hbm.at[pl.ds(cb, CHUNK)], idx_v.at[s4], isem[s4])
        pltpu.async_copy(seg_hbm.at[pl.ds(cb, CHUNK)],
                         segc_v.at[s4, pl.ds(0, CHUNK)], isem[s4])

    def wait_i(s4):
        pltpu.make_async_copy(src_hbm.at[pl.ds(0, CHUNK)], idx_v.at[s4],
                              isem[s4]).wait()
        pltpu.make_async_copy(seg_hbm.at[pl.ds(0, CHUNK)],
                              segc_v.at[s4, pl.ds(0, CHUNK)],
                              isem[s4]).wait()

    def issue_g(s4, b):
        pltpu.async_copy(word_hbm.at[idx_v.at[s4]], rows_v.at[b], gsem[b])

    def wait_g(s4, b):
        pltpu.make_async_copy(word_hbm.at[idx_v.at[s4]], rows_v.at[b],
                              gsem[b]).wait()

    def issue_o(c, b):
        cb = base + c * CHUNK
        pltpu.async_copy(rows_v.at[b], out_hbm.at[pl.ds(cb, CHUNK)], osem[b])

    def wait_o(b):
        pltpu.make_async_copy(rows_v.at[b], out_hbm.at[pl.ds(0, CHUNK)],
                              osem[b]).wait()

    def compute(c, b, s4):
        p0 = lax.rem(c * CHUNK, L)

        @plsc.parallel_loop(0, CHUNK, 16)
        def _grp(tb):
            sv = segc_v[s4, pl.ds(tb, 16)]
            for j in range(4):
                sums = []
                sqs = []
                for k in range(4):
                    t = tb + j * 4 + k
                    s_t = sv[j * 4 + k]
                    p_t = lax.rem(p0 + t, L)
                    ci = p_t * 3 + s_t
                    sum_v = None
                    sq_v = None
                    for f in range(NF):
                        d = pl.ds(f * 16, 16)
                        xf = rows_v[b, t, d] + combo_v[ci, d]
                        rows_v[b, t, d] = xf
                        sum_v = xf if sum_v is None else sum_v + xf
                        sq_v = xf * xf if sq_v is None else sq_v + xf * xf
                    sums.append(sum_v)
                    sqs.append(sq_v)
                means = []
                rstds = []
                for k in range(4):
                    mean = _allsum16(sums[k]) * (1.0 / EMB)
                    ex2 = _allsum16(sqs[k]) * (1.0 / EMB)
                    var = ex2 - mean * mean
                    means.append(mean)
                    rstds.append(_rsqrt16(var + 1e-6))
                for k in range(4):
                    t = tb + j * 4 + k
                    for f in range(NF):
                        d = pl.ds(f * 16, 16)
                        rg = rstds[k] * gamma[f]
                        off = beta[f] - means[k] * rg
                        rows_v[b, t, d] = rows_v[b, t, d] * rg + off

    issue_i(0, 0)
    issue_i(1, 1)
    wait_i(0)
    issue_g(0, 0)

    def iter4(c4, _):
        for u in range(4):
            b = u % 2
            nb = 1 - b
            c = c4 * 4 + u
            wait_g(u, b)

            @pl.when(c + 2 < NCHUNK)
            def _():
                issue_i(c + 2, (u + 2) % 4)

            @pl.when(c + 1 < NCHUNK)
            def _():
                @pl.when(c >= 1)
                def _():
                    wait_o(nb)
                wait_i((u + 1) % 4)
                issue_g((u + 1) % 4, nb)

            compute(c, b, u)
            issue_o(c, b)
        return 0

    lax.fori_loop(0, NCHUNK // 4, iter4, 0)
    wait_o(0)
    wait_o(1)


@jax.jit
def _tab_embedding(src, seg, word_table, seg_table, pos_table, gamma, beta):
    mesh = plsc.VectorSubcoreMesh(core_axis_name="c", subcore_axis_name="s")
    kern = pl.kernel(
        _body,
        out_type=jax.ShapeDtypeStruct((N, EMB), jnp.float32),
        mesh=mesh,
        scratch_types=[
            pltpu.VMEM((4, CHUNK), jnp.int32),
            pltpu.VMEM((4, CHUNK + 16), jnp.int32),
            pltpu.VMEM((2, CHUNK, EMB), jnp.float32),
            pltpu.VMEM((3 * L + 3, EMB), jnp.float32),
            pltpu.VMEM((2, EMB), jnp.float32),
            pltpu.SemaphoreType.DMA,
            pltpu.SemaphoreType.DMA,
            pltpu.SemaphoreType.DMA,
            pltpu.SemaphoreType.DMA,
            pltpu.SemaphoreType.DMA,
            pltpu.SemaphoreType.DMA,
            pltpu.SemaphoreType.DMA,
            pltpu.SemaphoreType.DMA,
        ],
    )
    out = kern(src.reshape(N), seg.reshape(N), word_table, seg_table,
               pos_table, gamma, beta)
    return out.reshape(B, L, EMB)


def kernel(src, seg, word_table, seg_table, pos_table, gamma, beta):
    return _tab_embedding(src, seg, word_table, seg_table, pos_table,
                          gamma, beta)
